# Initial kernel scaffold; baseline (speedup 1.0000x reference)
import jax
import jax.numpy as jnp
from jax.experimental import pallas as pl


def kernel(node_emb, edge_index, relation_emb, Ws1, bs1, Ws2, bs2, Wd1, bd1, Wd2, bd2, We1, be1, We2, be2):
    raise NotImplementedError("write your pallas kernel here")


# speedup vs baseline: 4.6557x; 4.6557x over previous
import functools

import jax
import jax.numpy as jnp
from jax import lax
from jax.experimental import pallas as pl
from jax.experimental.pallas import tpu as pltpu
from jax.experimental.pallas import tpu_sc as plsc

_NC = 2
_NS = 16
_NW = _NC * _NS
_LANES = 16


def _node_mlp_body(x_ref, w1_ref, b1_ref, w2_ref, b2_ref, o_ref):
    h = jnp.dot(x_ref[...], w1_ref[...], preferred_element_type=jnp.float32)
    h = jnp.maximum(h + b1_ref[...], 0.0)
    o_ref[...] = jnp.dot(h, w2_ref[...], preferred_element_type=jnp.float32) + b2_ref[...]


def _edge_mlp_body(x_ref, w1_ref, b1_ref, w2_ref, g_ref, o_ref):
    h = jnp.dot(x_ref[...], w1_ref[...], preferred_element_type=jnp.float32)
    h = jnp.maximum(h + b1_ref[...], 0.0)
    o_ref[...] = jnp.dot(h, w2_ref[...], preferred_element_type=jnp.float32) + g_ref[...]


def _node_tables(node_emb, w1cat, b1cat, w2cat, b2cat):
    n, d = node_emb.shape
    blk = 2000
    assert n % blk == 0
    h2 = w1cat.shape[1]
    return pl.pallas_call(
        _node_mlp_body,
        grid=(n // blk,),
        in_specs=[
            pl.BlockSpec((blk, d), lambda i: (i, 0)),
            pl.BlockSpec((d, h2), lambda i: (0, 0)),
            pl.BlockSpec((1, h2), lambda i: (0, 0)),
            pl.BlockSpec((h2, 2), lambda i: (0, 0)),
            pl.BlockSpec((1, 2), lambda i: (0, 0)),
        ],
        out_specs=pl.BlockSpec((blk, 2), lambda i: (i, 0)),
        out_shape=jax.ShapeDtypeStruct((n, 2), jnp.float32),
    )(node_emb, w1cat, b1cat, w2cat, b2cat)


def _edge_base(relation_emb, we1, be1, we2, gum):
    e, de = relation_emb.shape
    blk = 4000
    assert e % blk == 0
    h = we1.shape[1]
    return pl.pallas_call(
        _edge_mlp_body,
        grid=(e // blk,),
        in_specs=[
            pl.BlockSpec((blk, de), lambda i: (i, 0)),
            pl.BlockSpec((de, h), lambda i: (0, 0)),
            pl.BlockSpec((1, h), lambda i: (0, 0)),
            pl.BlockSpec((h, 1), lambda i: (0, 0)),
            pl.BlockSpec((blk, 1), lambda i: (i, 0)),
        ],
        out_specs=pl.BlockSpec((blk, 1), lambda i: (i, 0)),
        out_shape=jax.ShapeDtypeStruct((e, 1), jnp.float32),
    )(relation_emb, we1, be1, we2, gum)


def _sc_gather_sigmoid(wtab_flat, eidx_flat, ebase, inv_temp):
    e = ebase.shape[0]
    n2 = wtab_flat.shape[0]
    assert e % (_NW * _LANES) == 0
    chunk = e // _NW
    nvec = chunk // _LANES
    mesh = plsc.VectorSubcoreMesh(core_axis_name="c", subcore_axis_name="s")

    @functools.partial(
        pl.kernel,
        out_type=[
            jax.ShapeDtypeStruct((e,), jnp.float32),
            jax.ShapeDtypeStruct((_NW, _LANES), jnp.float32),
        ],
        mesh=mesh,
        compiler_params=pltpu.CompilerParams(needs_layout_passes=False),
        scratch_types=[
            pltpu.VMEM((n2,), jnp.float32),
            pltpu.VMEM((chunk,), jnp.int32),
            pltpu.VMEM((chunk,), jnp.int32),
            pltpu.VMEM((chunk,), jnp.float32),
            pltpu.VMEM((chunk,), jnp.float32),
            pltpu.VMEM((_LANES,), jnp.float32),
        ],
    )
    def sc_k(wtab_hbm, eidx_hbm, ebase_hbm, out_hbm, part_hbm,
             wtab_v, src_v, dst_v, eb_v, out_v, acc_v):
        wid = lax.axis_index("s") * _NC + lax.axis_index("c")
        base = wid * chunk
        pltpu.sync_copy(wtab_hbm, wtab_v)
        pltpu.sync_copy(eidx_hbm.at[pl.ds(base, chunk)], src_v)
        pltpu.sync_copy(eidx_hbm.at[pl.ds(e + base, chunk)], dst_v)
        pltpu.sync_copy(ebase_hbm.at[pl.ds(base, chunk)], eb_v)

        def body(i, acc):
            o = i * _LANES
            sidx = src_v[pl.ds(o, _LANES)]
            didx = dst_v[pl.ds(o, _LANES)]
            gs = plsc.load_gather(wtab_v, [sidx * 2])
            gd = plsc.load_gather(wtab_v, [didx * 2 + 1])
            x = (gs + gd + eb_v[pl.ds(o, _LANES)]) * inv_temp
            sig = 1.0 / (1.0 + jnp.exp(-x))
            out_v[pl.ds(o, _LANES)] = sig
            return acc + sig

        acc = lax.fori_loop(0, nvec, body, jnp.zeros((_LANES,), jnp.float32))
        acc_v[...] = acc
        pltpu.sync_copy(out_v, out_hbm.at[pl.ds(base, chunk)])
        pltpu.sync_copy(acc_v, part_hbm.at[wid])

    return sc_k(wtab_flat, eidx_flat, ebase)


def kernel(node_emb, edge_index, relation_emb, Ws1, bs1, Ws2, bs2,
           Wd1, bd1, Wd2, bd2, We1, be1, We2, be2):
    n, d = node_emb.shape
    e = edge_index.shape[1]
    h = Ws1.shape[1]
    temperature = 0.5

    w1cat = jnp.concatenate([Ws1, Wd1], axis=1)
    b1cat = jnp.concatenate([bs1, bd1]).reshape(1, 2 * h)
    zero2 = jnp.zeros_like(Ws2)
    w2cat = jnp.concatenate(
        [jnp.concatenate([Ws2, zero2], axis=1),
         jnp.concatenate([zero2, Wd2], axis=1)], axis=0)
    b2cat = jnp.concatenate([bs2, bd2]).reshape(1, 2)

    bias = 0.0001
    u = jax.random.uniform(jax.random.key(12345), (e,), jnp.float32)
    eps = (bias - (1.0 - bias)) * u + (1.0 - bias)
    gum = (jnp.log(eps) - jnp.log(1.0 - eps) + be2[0]).reshape(e, 1)

    wtab = _node_tables(node_emb, w1cat, b1cat, w2cat, b2cat)
    ebase = _edge_base(relation_emb, We1, be1.reshape(1, h), We2, gum)

    out, parts = _sc_gather_sigmoid(
        wtab.reshape(2 * n), edge_index.reshape(2 * e), ebase.reshape(e),
        inv_temp=1.0 / temperature)

    reg = 1.0 - parts.sum() / e
    return (reg, out.reshape(e, 1, 1))


# speedup vs baseline: 22.5063x; 4.8341x over previous
import functools

import jax
import jax.numpy as jnp
import numpy as np
from jax import lax
from jax.experimental import pallas as pl
from jax.experimental.pallas import tpu as pltpu
from jax.experimental.pallas import tpu_sc as plsc

_NC = 2
_NS = 16
_NW = _NC * _NS
_LANES = 16


def _rotl32(x, r):
    return ((x << np.uint32(r)) | (x >> np.uint32(32 - r))).astype(np.uint32)


def _threefry2x32(k0, k1, x0, x1):
    k0 = np.uint32(k0)
    k1 = np.uint32(k1)
    k2 = np.uint32(k0 ^ k1 ^ np.uint32(0x1BD11BDA))
    ks = (k0, k1, k2)
    x0 = (x0.astype(np.uint32) + k0).astype(np.uint32)
    x1 = (x1.astype(np.uint32) + k1).astype(np.uint32)
    for r in range(5):
        for rot in ((13, 15, 26, 6) if r % 2 == 0 else (17, 29, 16, 24)):
            x0 = (x0 + x1).astype(np.uint32)
            x1 = _rotl32(x1, rot)
            x1 = (x0 ^ x1).astype(np.uint32)
        x0 = (x0 + ks[(r + 1) % 3]).astype(np.uint32)
        x1 = (x1 + ks[(r + 2) % 3] + np.uint32(r + 1)).astype(np.uint32)
    return x0, x1


_GUM_CACHE = {}


def _gumbel_const(e):
    if e not in _GUM_CACHE:
        i = np.arange(e, dtype=np.uint64)
        hi = (i >> np.uint64(32)).astype(np.uint32)
        lo = (i & np.uint64(0xFFFFFFFF)).astype(np.uint32)
        b0, b1 = _threefry2x32(0, 12345, hi, lo)
        bits = b0 ^ b1
        u = ((bits >> np.uint32(9)) | np.uint32(0x3F800000)).view(np.float32) \
            - np.float32(1.0)
        bias = np.float32(0.0001)
        one = np.float32(1.0)
        eps = (bias - (one - bias)) * u + (one - bias)
        _GUM_CACHE[e] = np.log(eps) - np.log(one - eps)
    return _GUM_CACHE[e]


def _node_mlp_body(x_ref, w1t_ref, b1_ref, w2t_ref, b2_ref, o_ref):
    ht = lax.dot_general(w1t_ref[...], x_ref[...], (((1,), (1,)), ((), ())),
                         preferred_element_type=jnp.float32)
    ht = jnp.maximum(ht + b1_ref[...], 0.0)
    o_ref[...] = lax.dot_general(w2t_ref[...], ht, (((1,), (0,)), ((), ())),
                                 preferred_element_type=jnp.float32) + b2_ref[...]


def _edge_mlp_body(x_ref, w1t_ref, b1_ref, w2t_ref, b2_ref, g_ref, o_ref):
    ht = lax.dot_general(w1t_ref[...], x_ref[...], (((1,), (1,)), ((), ())),
                         preferred_element_type=jnp.float32)
    ht = jnp.maximum(ht + b1_ref[...], 0.0)
    row = (lax.dot_general(w2t_ref[...], ht, (((1,), (0,)), ((), ())),
                           preferred_element_type=jnp.float32)
           + b2_ref[...])
    o_ref[...] = row.reshape(1, 1, row.shape[1]) + g_ref[...]


def _node_tables(node_emb, w1t, b1col, w2t, b2col):
    n, d = node_emb.shape
    h2 = w1t.shape[0]
    return pl.pallas_call(
        _node_mlp_body,
        out_shape=jax.ShapeDtypeStruct((2, n), jnp.float32),
    )(node_emb, w1t, b1col, w2t, b2col)


_EDGE_BLK = 32000


def _edge_base(relation_emb, w1t, b1col, w2t, b2col, gum):
    e, de = relation_emb.shape
    blk = _EDGE_BLK
    assert e % blk == 0
    grid = e // blk
    h = w1t.shape[0]
    return pl.pallas_call(
        _edge_mlp_body,
        grid=(grid,),
        in_specs=[
            pl.BlockSpec((blk, de), lambda i: (i, 0)),
            pl.BlockSpec((h, de), lambda i: (0, 0)),
            pl.BlockSpec((h, 1), lambda i: (0, 0)),
            pl.BlockSpec((1, h), lambda i: (0, 0)),
            pl.BlockSpec((1, 1), lambda i: (0, 0)),
            pl.BlockSpec((1, 1, blk), lambda i: (i, 0, 0)),
        ],
        out_specs=pl.BlockSpec((1, 1, blk), lambda i: (i, 0, 0)),
        out_shape=jax.ShapeDtypeStruct((grid, 1, blk), jnp.float32),
    )(relation_emb, w1t, b1col, w2t, b2col, gum.reshape(grid, 1, blk))


def _sc_gather_sigmoid(wtab_flat, edge_index, ebase, inv_temp, n):
    e = ebase.shape[0]
    n2 = wtab_flat.shape[0]
    assert e % (_NW * _LANES) == 0
    chunk = e // _NW
    nvec = chunk // _LANES
    mesh = plsc.VectorSubcoreMesh(core_axis_name="c", subcore_axis_name="s")

    @functools.partial(
        pl.kernel,
        out_type=[
            jax.ShapeDtypeStruct((e,), jnp.float32),
            jax.ShapeDtypeStruct((_NW, _LANES), jnp.float32),
        ],
        mesh=mesh,
        compiler_params=pltpu.CompilerParams(needs_layout_passes=False),
        scratch_types=[
            pltpu.VMEM((n2,), jnp.float32),
            pltpu.VMEM((chunk,), jnp.int32),
            pltpu.VMEM((chunk,), jnp.int32),
            pltpu.VMEM((chunk,), jnp.float32),
            pltpu.VMEM((chunk,), jnp.float32),
            pltpu.VMEM((_LANES,), jnp.float32),
        ],
    )
    def sc_k(wtab_hbm, eidx_hbm, ebase_hbm, out_hbm, part_hbm,
             wtab_v, src_v, dst_v, eb_v, out_v, acc_v):
        wid = lax.axis_index("s") * _NC + lax.axis_index("c")
        base = wid * chunk
        pltpu.sync_copy(wtab_hbm, wtab_v)
        pltpu.sync_copy(eidx_hbm.at[pl.ds(base, chunk)], src_v)
        pltpu.sync_copy(eidx_hbm.at[pl.ds(e + base, chunk)], dst_v)
        pltpu.sync_copy(ebase_hbm.at[pl.ds(base, chunk)], eb_v)

        def body(i, acc):
            o = i * _LANES
            sidx = src_v[pl.ds(o, _LANES)]
            didx = dst_v[pl.ds(o, _LANES)]
            gs = plsc.load_gather(wtab_v, [sidx])
            gd = plsc.load_gather(wtab_v, [didx + n])
            x = (gs + gd + eb_v[pl.ds(o, _LANES)]) * inv_temp
            sig = 1.0 / (1.0 + jnp.exp(-x))
            out_v[pl.ds(o, _LANES)] = sig
            return acc + sig

        acc = lax.fori_loop(0, nvec, body, jnp.zeros((_LANES,), jnp.float32))
        acc_v[...] = acc
        pltpu.sync_copy(out_v, out_hbm.at[pl.ds(base, chunk)])
        pltpu.sync_copy(acc_v, part_hbm.at[wid])

    return sc_k(wtab_flat, edge_index, ebase)


def kernel(node_emb, edge_index, relation_emb, Ws1, bs1, Ws2, bs2,
           Wd1, bd1, Wd2, bd2, We1, be1, We2, be2):
    n, d = node_emb.shape
    e = edge_index.shape[1]
    h = Ws1.shape[1]
    temperature = 0.5

    w1t = jnp.concatenate([Ws1, Wd1], axis=1).T
    b1col = jnp.concatenate([bs1, bd1]).reshape(2 * h, 1)
    zero2 = jnp.zeros_like(Ws2)
    w2t = jnp.concatenate(
        [jnp.concatenate([Ws2, zero2], axis=1),
         jnp.concatenate([zero2, Wd2], axis=1)], axis=0).T
    b2col = jnp.concatenate([bs2, bd2]).reshape(2, 1)

    gum = jnp.asarray(_gumbel_const(e))

    wtab2 = _node_tables(node_emb, w1t, b1col, w2t, b2col)
    ebase = _edge_base(relation_emb, We1.T, be1.reshape(h, 1),
                       We2.T, be2.reshape(1, 1), gum)

    out, parts = _sc_gather_sigmoid(
        wtab2.reshape(2 * n), edge_index.reshape(2 * e), ebase.reshape(e),
        inv_temp=1.0 / temperature, n=n)

    reg = 1.0 - parts.sum() / e
    return (reg, out.reshape(e, 1, 1))


# speedup vs baseline: 22.7263x; 1.0098x over previous
import functools

import jax
import jax.numpy as jnp
import numpy as np
from jax import lax
from jax.experimental import pallas as pl
from jax.experimental.pallas import tpu as pltpu
from jax.experimental.pallas import tpu_sc as plsc

_NC = 2
_NS = 16
_NW = _NC * _NS
_LANES = 16


def _rotl32(x, r):
    return ((x << np.uint32(r)) | (x >> np.uint32(32 - r))).astype(np.uint32)


def _threefry2x32(k0, k1, x0, x1):
    k0 = np.uint32(k0)
    k1 = np.uint32(k1)
    k2 = np.uint32(k0 ^ k1 ^ np.uint32(0x1BD11BDA))
    ks = (k0, k1, k2)
    x0 = (x0.astype(np.uint32) + k0).astype(np.uint32)
    x1 = (x1.astype(np.uint32) + k1).astype(np.uint32)
    for r in range(5):
        for rot in ((13, 15, 26, 6) if r % 2 == 0 else (17, 29, 16, 24)):
            x0 = (x0 + x1).astype(np.uint32)
            x1 = _rotl32(x1, rot)
            x1 = (x0 ^ x1).astype(np.uint32)
        x0 = (x0 + ks[(r + 1) % 3]).astype(np.uint32)
        x1 = (x1 + ks[(r + 2) % 3] + np.uint32(r + 1)).astype(np.uint32)
    return x0, x1


_GUM_CACHE = {}


def _gumbel_const(e):
    if e not in _GUM_CACHE:
        i = np.arange(e, dtype=np.uint64)
        hi = (i >> np.uint64(32)).astype(np.uint32)
        lo = (i & np.uint64(0xFFFFFFFF)).astype(np.uint32)
        b0, b1 = _threefry2x32(0, 12345, hi, lo)
        bits = b0 ^ b1
        u = ((bits >> np.uint32(9)) | np.uint32(0x3F800000)).view(np.float32) \
            - np.float32(1.0)
        bias = np.float32(0.0001)
        one = np.float32(1.0)
        eps = (bias - (one - bias)) * u + (one - bias)
        _GUM_CACHE[e] = np.log(eps) - np.log(one - eps)
    return _GUM_CACHE[e]


def _node_mlp_body(x_ref, w1t_ref, b1_ref, w2t_ref, b2_ref, o_ref):
    ht = lax.dot_general(w1t_ref[...], x_ref[...], (((1,), (1,)), ((), ())),
                         preferred_element_type=jnp.float32)
    ht = jnp.maximum(ht + b1_ref[...], 0.0)
    o_ref[...] = lax.dot_general(w2t_ref[...], ht, (((1,), (0,)), ((), ())),
                                 preferred_element_type=jnp.float32) + b2_ref[...]


def _edge_mlp_body(x_ref, w1t_ref, b1_ref, w2t_ref, b2_ref, g_ref, o_ref):
    ht = lax.dot_general(w1t_ref[...], x_ref[...], (((1,), (1,)), ((), ())),
                         preferred_element_type=jnp.float32)
    ht = jnp.maximum(ht + b1_ref[...], 0.0)
    row = (lax.dot_general(w2t_ref[...], ht, (((1,), (0,)), ((), ())),
                           preferred_element_type=jnp.float32)
           + b2_ref[...])
    o_ref[...] = row.reshape(1, 1, row.shape[1]) + g_ref[...]


def _node_tables(node_emb, w1t, b1col, w2t, b2col):
    n, d = node_emb.shape
    h2 = w1t.shape[0]
    return pl.pallas_call(
        _node_mlp_body,
        out_shape=jax.ShapeDtypeStruct((2, n), jnp.float32),
    )(node_emb, w1t, b1col, w2t, b2col)


_EDGE_BLK = 32000


def _edge_base(relation_emb, w1t, b1col, w2t, b2col, gum):
    e, de = relation_emb.shape
    blk = _EDGE_BLK
    assert e % blk == 0
    grid = e // blk
    h = w1t.shape[0]
    return pl.pallas_call(
        _edge_mlp_body,
        grid=(grid,),
        in_specs=[
            pl.BlockSpec((blk, de), lambda i: (i, 0)),
            pl.BlockSpec((h, de), lambda i: (0, 0)),
            pl.BlockSpec((h, 1), lambda i: (0, 0)),
            pl.BlockSpec((1, h), lambda i: (0, 0)),
            pl.BlockSpec((1, 1), lambda i: (0, 0)),
            pl.BlockSpec((1, 1, blk), lambda i: (i, 0, 0)),
        ],
        out_specs=pl.BlockSpec((1, 1, blk), lambda i: (i, 0, 0)),
        out_shape=jax.ShapeDtypeStruct((grid, 1, blk), jnp.float32),
    )(relation_emb, w1t, b1col, w2t, b2col, gum.reshape(grid, 1, blk))


_ALIGN = 128


def _sc_gather_sigmoid(wtab_flat, edge_index, ebase, inv_temp, n):
    e = ebase.shape[0]
    n2 = wtab_flat.shape[0]
    assert e % (_NW * _LANES) == 0
    chunk = e // _NW
    nvec = chunk // _LANES
    asz = -(-chunk // _ALIGN) * _ALIGN
    assert asz - chunk >= (_ALIGN - _LANES) % _ALIGN or chunk % _ALIGN == 0
    mesh = plsc.VectorSubcoreMesh(core_axis_name="c", subcore_axis_name="s")

    @functools.partial(
        pl.kernel,
        out_type=[
            jax.ShapeDtypeStruct((e,), jnp.float32),
            jax.ShapeDtypeStruct((_NW, _LANES), jnp.float32),
        ],
        mesh=mesh,
        compiler_params=pltpu.CompilerParams(needs_layout_passes=False),
        scratch_types=[
            pltpu.VMEM((n2,), jnp.float32),
            pltpu.VMEM((2, asz), jnp.int32),
            pltpu.VMEM((chunk,), jnp.float32),
            pltpu.VMEM((chunk,), jnp.float32),
            pltpu.VMEM((_LANES,), jnp.float32),
        ],
    )
    def sc_k(wtab_hbm, eidx_hbm, ebase_hbm, out_hbm, part_hbm,
             wtab_v, eidx_v, eb_v, out_v, acc_v):
        wid = lax.axis_index("s") * _NC + lax.axis_index("c")
        base = wid * chunk
        off = lax.rem(base, _ALIGN)
        abase = pl.multiple_of(base - off, _ALIGN)
        pltpu.sync_copy(wtab_hbm, wtab_v)
        pltpu.sync_copy(eidx_hbm.at[:, pl.ds(abase, asz)], eidx_v)
        pltpu.sync_copy(ebase_hbm.at[pl.ds(base, chunk)], eb_v)

        def body(i, acc):
            o = off + i * _LANES
            sidx = eidx_v[0, pl.ds(o, _LANES)]
            didx = eidx_v[1, pl.ds(o, _LANES)]
            gs = plsc.load_gather(wtab_v, [sidx])
            gd = plsc.load_gather(wtab_v, [didx + n])
            x = (gs + gd + eb_v[pl.ds(i * _LANES, _LANES)]) * inv_temp
            sig = 1.0 / (1.0 + jnp.exp(-x))
            out_v[pl.ds(i * _LANES, _LANES)] = sig
            return acc + sig

        acc = lax.fori_loop(0, nvec, body, jnp.zeros((_LANES,), jnp.float32))
        acc_v[...] = acc
        pltpu.sync_copy(out_v, out_hbm.at[pl.ds(base, chunk)])
        pltpu.sync_copy(acc_v, part_hbm.at[wid])

    return sc_k(wtab_flat, edge_index, ebase)


def kernel(node_emb, edge_index, relation_emb, Ws1, bs1, Ws2, bs2,
           Wd1, bd1, Wd2, bd2, We1, be1, We2, be2):
    n, d = node_emb.shape
    e = edge_index.shape[1]
    h = Ws1.shape[1]
    temperature = 0.5

    w1t = jnp.concatenate([Ws1, Wd1], axis=1).T
    b1col = jnp.concatenate([bs1, bd1]).reshape(2 * h, 1)
    zero2 = jnp.zeros_like(Ws2)
    w2t = jnp.concatenate(
        [jnp.concatenate([Ws2, zero2], axis=1),
         jnp.concatenate([zero2, Wd2], axis=1)], axis=0).T
    b2col = jnp.concatenate([bs2, bd2]).reshape(2, 1)

    gum = jnp.asarray(_gumbel_const(e))

    wtab2 = _node_tables(node_emb, w1t, b1col, w2t, b2col)
    ebase = _edge_base(relation_emb, We1.T, be1.reshape(h, 1),
                       We2.T, be2.reshape(1, 1), gum)

    out, parts = _sc_gather_sigmoid(
        wtab2.reshape(2 * n), edge_index, ebase.reshape(e),
        inv_temp=1.0 / temperature, n=n)

    reg = 1.0 - parts.sum() / e
    return (reg, out.reshape(e, 1, 1))


# speedup vs baseline: 26.4454x; 1.1636x over previous
import functools

import jax
import jax.numpy as jnp
import numpy as np
from jax import lax
from jax.experimental import pallas as pl
from jax.experimental.pallas import tpu as pltpu
from jax.experimental.pallas import tpu_sc as plsc

_NC = 2
_NS = 16
_NW = _NC * _NS
_LANES = 16


def _rotl32(x, r):
    return ((x << np.uint32(r)) | (x >> np.uint32(32 - r))).astype(np.uint32)


def _threefry2x32(k0, k1, x0, x1):
    k0 = np.uint32(k0)
    k1 = np.uint32(k1)
    k2 = np.uint32(k0 ^ k1 ^ np.uint32(0x1BD11BDA))
    ks = (k0, k1, k2)
    x0 = (x0.astype(np.uint32) + k0).astype(np.uint32)
    x1 = (x1.astype(np.uint32) + k1).astype(np.uint32)
    for r in range(5):
        for rot in ((13, 15, 26, 6) if r % 2 == 0 else (17, 29, 16, 24)):
            x0 = (x0 + x1).astype(np.uint32)
            x1 = _rotl32(x1, rot)
            x1 = (x0 ^ x1).astype(np.uint32)
        x0 = (x0 + ks[(r + 1) % 3]).astype(np.uint32)
        x1 = (x1 + ks[(r + 2) % 3] + np.uint32(r + 1)).astype(np.uint32)
    return x0, x1


_GUM_CACHE = {}


def _gumbel_const(e):
    if e not in _GUM_CACHE:
        i = np.arange(e, dtype=np.uint64)
        hi = (i >> np.uint64(32)).astype(np.uint32)
        lo = (i & np.uint64(0xFFFFFFFF)).astype(np.uint32)
        b0, b1 = _threefry2x32(0, 12345, hi, lo)
        bits = b0 ^ b1
        u = ((bits >> np.uint32(9)) | np.uint32(0x3F800000)).view(np.float32) \
            - np.float32(1.0)
        bias = np.float32(0.0001)
        one = np.float32(1.0)
        eps = (bias - (one - bias)) * u + (one - bias)
        _GUM_CACHE[e] = np.log(eps) - np.log(one - eps)
    return _GUM_CACHE[e]


def _node_mlp_body(x_ref, w1t_ref, b1_ref, w2t_ref, b2_ref, o_ref):
    ht = lax.dot_general(w1t_ref[...], x_ref[...], (((1,), (1,)), ((), ())),
                         preferred_element_type=jnp.float32)
    ht = jnp.maximum(ht + b1_ref[...], 0.0)
    o_ref[...] = lax.dot_general(w2t_ref[...], ht, (((1,), (0,)), ((), ())),
                                 preferred_element_type=jnp.float32) + b2_ref[...]


def _node_tables(node_emb, w1t, b1col, w2t, b2col):
    n, d = node_emb.shape
    return pl.pallas_call(
        _node_mlp_body,
        out_shape=jax.ShapeDtypeStruct((2, n), jnp.float32),
    )(node_emb, w1t, b1col, w2t, b2col)


def _edge_mlp_body(inv_temp, x_ref, w1t_ref, b1_ref, w2t_ref, b2_ref,
                   g_ref, ws_ref, o_ref, p_ref):
    ht = lax.dot_general(w1t_ref[...], x_ref[...], (((1,), (1,)), ((), ())),
                         preferred_element_type=jnp.float32)
    ht = jnp.maximum(ht + b1_ref[...], 0.0)
    row = (lax.dot_general(w2t_ref[...], ht, (((1,), (0,)), ((), ())),
                           preferred_element_type=jnp.float32)
           + b2_ref[...])
    x = (row.reshape(1, 1, row.shape[1]) + g_ref[...] + ws_ref[...]) * inv_temp
    sig = 1.0 / (1.0 + jnp.exp(-x))
    o_ref[...] = sig
    p_ref[...] = jnp.broadcast_to(jnp.sum(sig), p_ref.shape)


_EDGE_BLK = 32000


def _edge_finale(relation_emb, w1t, b1col, w2t, b2, gum3, wsum3, inv_temp):
    e, de = relation_emb.shape
    blk = _EDGE_BLK
    assert e % blk == 0
    grid = e // blk
    h = w1t.shape[0]
    return pl.pallas_call(
        functools.partial(_edge_mlp_body, inv_temp),
        grid=(grid,),
        in_specs=[
            pl.BlockSpec((blk, de), lambda i: (i, 0)),
            pl.BlockSpec((h, de), lambda i: (0, 0)),
            pl.BlockSpec((h, 1), lambda i: (0, 0)),
            pl.BlockSpec((1, h), lambda i: (0, 0)),
            pl.BlockSpec((1, 1), lambda i: (0, 0)),
            pl.BlockSpec((1, 1, blk), lambda i: (i, 0, 0)),
            pl.BlockSpec((1, 1, blk), lambda i: (i, 0, 0)),
        ],
        out_specs=[
            pl.BlockSpec((1, 1, blk), lambda i: (i, 0, 0)),
            pl.BlockSpec((1, 1, 128), lambda i: (i, 0, 0)),
        ],
        out_shape=[
            jax.ShapeDtypeStruct((grid, 1, blk), jnp.float32),
            jax.ShapeDtypeStruct((grid, 1, 128), jnp.float32),
        ],
    )(relation_emb, w1t, b1col, w2t, b2, gum3, wsum3)


_ALIGN = 128


def _sc_gather(wtab_flat, edge_index, n):
    e = edge_index.shape[1]
    n2 = wtab_flat.shape[0]
    assert e % (_NW * _LANES) == 0
    chunk = e // _NW
    nvec = chunk // _LANES
    asz = -(-chunk // _ALIGN) * _ALIGN
    mesh = plsc.VectorSubcoreMesh(core_axis_name="c", subcore_axis_name="s")

    @functools.partial(
        pl.kernel,
        out_type=jax.ShapeDtypeStruct((e,), jnp.float32),
        mesh=mesh,
        compiler_params=pltpu.CompilerParams(needs_layout_passes=False),
        scratch_types=[
            pltpu.VMEM((n2,), jnp.float32),
            pltpu.VMEM((2, asz), jnp.int32),
            pltpu.VMEM((chunk,), jnp.float32),
        ],
    )
    def sc_k(wtab_hbm, eidx_hbm, out_hbm, wtab_v, eidx_v, out_v):
        wid = lax.axis_index("s") * _NC + lax.axis_index("c")
        base = wid * chunk
        off = lax.rem(base, _ALIGN)
        abase = pl.multiple_of(base - off, _ALIGN)
        pltpu.sync_copy(wtab_hbm, wtab_v)
        pltpu.sync_copy(eidx_hbm.at[:, pl.ds(abase, asz)], eidx_v)

        def body(i, carry):
            o = off + i * _LANES
            sidx = eidx_v[0, pl.ds(o, _LANES)]
            didx = eidx_v[1, pl.ds(o, _LANES)]
            gs = plsc.load_gather(wtab_v, [sidx])
            gd = plsc.load_gather(wtab_v, [didx + n])
            out_v[pl.ds(i * _LANES, _LANES)] = gs + gd
            return carry

        lax.fori_loop(0, nvec, body, 0)
        pltpu.sync_copy(out_v, out_hbm.at[pl.ds(base, chunk)])

    return sc_k(wtab_flat, edge_index)


def kernel(node_emb, edge_index, relation_emb, Ws1, bs1, Ws2, bs2,
           Wd1, bd1, Wd2, bd2, We1, be1, We2, be2):
    n, d = node_emb.shape
    e = edge_index.shape[1]
    h = Ws1.shape[1]
    temperature = 0.5
    blk = _EDGE_BLK
    grid = e // blk

    w1t = jnp.concatenate([Ws1, Wd1], axis=1).T
    b1col = jnp.concatenate([bs1, bd1]).reshape(2 * h, 1)
    zero2 = jnp.zeros_like(Ws2)
    w2t = jnp.concatenate(
        [jnp.concatenate([Ws2, zero2], axis=1),
         jnp.concatenate([zero2, Wd2], axis=1)], axis=0).T
    b2col = jnp.concatenate([bs2, bd2]).reshape(2, 1)

    gum3 = jnp.asarray(_gumbel_const(e).reshape(grid, 1, blk))

    wtab2 = _node_tables(node_emb, w1t, b1col, w2t, b2col)
    wsum = _sc_gather(wtab2.reshape(2 * n), edge_index, n)

    out3, parts = _edge_finale(
        relation_emb, We1.T, be1.reshape(h, 1), We2.T, be2.reshape(1, 1),
        gum3, wsum.reshape(grid, 1, blk), inv_temp=1.0 / temperature)

    reg = 1.0 - parts[:, 0, 0].sum() / e
    return (reg, out3.reshape(e, 1, 1))


# speedup vs baseline: 58.4408x; 2.2099x over previous
import functools

import jax
import jax.numpy as jnp
import numpy as np
from jax import lax
from jax.experimental import pallas as pl
from jax.experimental.pallas import tpu as pltpu
from jax.experimental.pallas import tpu_sc as plsc

_NC = 2
_NS = 16
_NW = _NC * _NS
_LANES = 16


def _rotl32(x, r):
    return ((x << np.uint32(r)) | (x >> np.uint32(32 - r))).astype(np.uint32)


def _threefry2x32(k0, k1, x0, x1):
    k0 = np.uint32(k0)
    k1 = np.uint32(k1)
    k2 = np.uint32(k0 ^ k1 ^ np.uint32(0x1BD11BDA))
    ks = (k0, k1, k2)
    x0 = (x0.astype(np.uint32) + k0).astype(np.uint32)
    x1 = (x1.astype(np.uint32) + k1).astype(np.uint32)
    for r in range(5):
        for rot in ((13, 15, 26, 6) if r % 2 == 0 else (17, 29, 16, 24)):
            x0 = (x0 + x1).astype(np.uint32)
            x1 = _rotl32(x1, rot)
            x1 = (x0 ^ x1).astype(np.uint32)
        x0 = (x0 + ks[(r + 1) % 3]).astype(np.uint32)
        x1 = (x1 + ks[(r + 2) % 3] + np.uint32(r + 1)).astype(np.uint32)
    return x0, x1


_GUM_CACHE = {}


def _gumbel_const(e):
    if e not in _GUM_CACHE:
        i = np.arange(e, dtype=np.uint64)
        hi = (i >> np.uint64(32)).astype(np.uint32)
        lo = (i & np.uint64(0xFFFFFFFF)).astype(np.uint32)
        b0, b1 = _threefry2x32(0, 12345, hi, lo)
        bits = b0 ^ b1
        u = ((bits >> np.uint32(9)) | np.uint32(0x3F800000)).view(np.float32) \
            - np.float32(1.0)
        bias = np.float32(0.0001)
        one = np.float32(1.0)
        eps = (bias - (one - bias)) * u + (one - bias)
        _GUM_CACHE[e] = np.log(eps) - np.log(one - eps)
    return _GUM_CACHE[e]


def _node_mlp_body(x_ref, w1t_ref, b1_ref, w2t_ref, b2_ref, o_ref):
    ht = lax.dot_general(w1t_ref[...], x_ref[...], (((1,), (1,)), ((), ())),
                         preferred_element_type=jnp.float32)
    ht = jnp.maximum(ht + b1_ref[...], 0.0)
    o_ref[...] = lax.dot_general(w2t_ref[...], ht, (((1,), (0,)), ((), ())),
                                 preferred_element_type=jnp.float32) + b2_ref[...]


def _node_tables(node_emb, w1t, b1col, w2t, b2col):
    n, d = node_emb.shape
    return pl.pallas_call(
        _node_mlp_body,
        out_shape=jax.ShapeDtypeStruct((2, n), jnp.float32),
    )(node_emb, w1t, b1col, w2t, b2col)


def _edge_mlp_body(inv_temp, xt_ref, w1t_ref, b1_ref, w2t_ref, b2_ref,
                   g_ref, ws_ref, o_ref, p_ref):
    ht = lax.dot_general(w1t_ref[...], xt_ref[...], (((1,), (0,)), ((), ())),
                         preferred_element_type=jnp.float32)
    ht = jnp.maximum(ht + b1_ref[...], 0.0)
    row = (lax.dot_general(w2t_ref[...], ht, (((1,), (0,)), ((), ())),
                           preferred_element_type=jnp.float32)
           + b2_ref[...])
    x = (row.reshape(1, 1, row.shape[1]) + g_ref[...] + ws_ref[...]) * inv_temp
    sig = 1.0 / (1.0 + jnp.exp(-x))
    o_ref[...] = sig
    p_ref[...] = jnp.broadcast_to(jnp.sum(sig), p_ref.shape)


_EDGE_BLK = 12800


def _edge_finale(relation_t, w1t, b1col, w2t, b2, gum3, wsum3, inv_temp):
    de, e = relation_t.shape
    blk = _EDGE_BLK
    assert e % blk == 0
    grid = e // blk
    h = w1t.shape[0]
    return pl.pallas_call(
        functools.partial(_edge_mlp_body, inv_temp),
        grid=(grid,),
        in_specs=[
            pl.BlockSpec((de, blk), lambda i: (0, i)),
            pl.BlockSpec((h, de), lambda i: (0, 0)),
            pl.BlockSpec((h, 1), lambda i: (0, 0)),
            pl.BlockSpec((1, h), lambda i: (0, 0)),
            pl.BlockSpec((1, 1), lambda i: (0, 0)),
            pl.BlockSpec((1, 1, blk), lambda i: (i, 0, 0)),
            pl.BlockSpec((1, 1, blk), lambda i: (i, 0, 0)),
        ],
        out_specs=[
            pl.BlockSpec((1, 1, blk), lambda i: (i, 0, 0)),
            pl.BlockSpec((1, 1, 128), lambda i: (i, 0, 0)),
        ],
        out_shape=[
            jax.ShapeDtypeStruct((grid, 1, blk), jnp.float32),
            jax.ShapeDtypeStruct((grid, 1, 128), jnp.float32),
        ],
    )(relation_t, w1t, b1col, w2t, b2, gum3, wsum3)


_ALIGN = 128


def _sc_gather(wtab_flat, edge_index, n):
    e = edge_index.shape[1]
    n2 = wtab_flat.shape[0]
    assert e % (_NW * _LANES) == 0
    chunk = e // _NW
    nvec = chunk // _LANES
    asz = -(-chunk // _ALIGN) * _ALIGN
    mesh = plsc.VectorSubcoreMesh(core_axis_name="c", subcore_axis_name="s")

    @functools.partial(
        pl.kernel,
        out_type=jax.ShapeDtypeStruct((e,), jnp.float32),
        mesh=mesh,
        compiler_params=pltpu.CompilerParams(needs_layout_passes=False),
        scratch_types=[
            pltpu.VMEM((n2,), jnp.float32),
            pltpu.VMEM((2, asz), jnp.int32),
            pltpu.VMEM((chunk,), jnp.float32),
        ],
    )
    def sc_k(wtab_hbm, eidx_hbm, out_hbm, wtab_v, eidx_v, out_v):
        wid = lax.axis_index("s") * _NC + lax.axis_index("c")
        base = wid * chunk
        off = lax.rem(base, _ALIGN)
        abase = pl.multiple_of(base - off, _ALIGN)
        pltpu.sync_copy(wtab_hbm, wtab_v)
        pltpu.sync_copy(eidx_hbm.at[:, pl.ds(abase, asz)], eidx_v)

        def body(i, carry):
            o = off + i * _LANES
            sidx = eidx_v[0, pl.ds(o, _LANES)]
            didx = eidx_v[1, pl.ds(o, _LANES)]
            gs = plsc.load_gather(wtab_v, [sidx])
            gd = plsc.load_gather(wtab_v, [didx + n])
            out_v[pl.ds(i * _LANES, _LANES)] = gs + gd
            return carry

        lax.fori_loop(0, nvec, body, 0)
        pltpu.sync_copy(out_v, out_hbm.at[pl.ds(base, chunk)])

    return sc_k(wtab_flat, edge_index)


def kernel(node_emb, edge_index, relation_emb, Ws1, bs1, Ws2, bs2,
           Wd1, bd1, Wd2, bd2, We1, be1, We2, be2):
    n, d = node_emb.shape
    e = edge_index.shape[1]
    h = Ws1.shape[1]
    temperature = 0.5
    blk = _EDGE_BLK
    grid = e // blk

    w1t = jnp.concatenate([Ws1, Wd1], axis=1).T
    b1col = jnp.concatenate([bs1, bd1]).reshape(2 * h, 1)
    zero2 = jnp.zeros_like(Ws2)
    w2t = jnp.concatenate(
        [jnp.concatenate([Ws2, zero2], axis=1),
         jnp.concatenate([zero2, Wd2], axis=1)], axis=0).T
    b2col = jnp.concatenate([bs2, bd2]).reshape(2, 1)

    gum3 = jnp.asarray(_gumbel_const(e).reshape(grid, 1, blk))

    wtab2 = _node_tables(node_emb, w1t, b1col, w2t, b2col)
    wsum = _sc_gather(wtab2.reshape(2 * n), edge_index, n)

    out3, parts = _edge_finale(
        relation_emb.T, We1.T, be1.reshape(h, 1), We2.T, be2.reshape(1, 1),
        gum3, wsum.reshape(grid, 1, blk), inv_temp=1.0 / temperature)

    reg = 1.0 - parts[:, 0, 0].sum() / e
    return (reg, out3.reshape(e, 1, 1))
